# baseline (speedup 1.0000x reference)
import functools

import jax
import jax.numpy as jnp
from jax import lax
from jax.experimental import pallas as pl
from jax.experimental.pallas import tpu as pltpu
from jax.experimental.pallas import tpu_sc as plsc

NC = 2
NS = 16
NW = NC * NS

VOCAB = 100000
DIM = 64
B = 16384
L = 10
NEG = 5

CHUNK = 128
ROWS_PER_W = B // NW
NCHUNK = ROWS_PER_W // CHUNK
K16 = DIM // 16


def _sc_body(ctx_flat, word, neg_flat, emb0, emb1, pos_hbm, neg_hbm,
             cidx_v, clist_v, widx_v, nidx_v, csum_v, w_v, neg_v,
             pos_o, neg_o, sem_c, sem_w, sem_n):
    wid = lax.axis_index("s") * NC + lax.axis_index("c")
    lane = lax.iota(jnp.int32, 16)
    laneL = lane * L
    for chunk in range(NCHUNK):
        base = wid * ROWS_PER_W + chunk * CHUNK
        pltpu.sync_copy(ctx_flat.at[pl.ds(base * L, CHUNK * L)], cidx_v)
        pltpu.sync_copy(word.at[pl.ds(base, CHUNK)], widx_v)
        pltpu.sync_copy(neg_flat.at[pl.ds(base * NEG, CHUNK * NEG)], nidx_v)

        for l in range(L):
            for s in range(CHUNK // 16):
                v = plsc.load_gather(cidx_v, [laneL + (s * 16 * L + l)])
                clist_v[l, pl.ds(s * 16, 16)] = v

        d_w = pltpu.async_copy(emb1.at[widx_v], w_v, sem_w)
        d_n = [pltpu.async_copy(emb1.at[nidx_v.at[pl.ds(j * CHUNK, CHUNK)]],
                                neg_v.at[pl.ds(j * CHUNK, CHUNK)], sem_n)
               for j in range(NEG)]
        pltpu.async_copy(emb0.at[clist_v.at[0]], csum_v, sem_c).wait()
        d_c = [pltpu.async_copy(emb0.at[clist_v.at[l]], csum_v, sem_c,
                                add=True)
               for l in range(1, L)]
        for d in d_c:
            d.wait()
        d_w.wait()
        for d in d_n:
            d.wait()

        zero = jnp.zeros((16,), jnp.float32)
        for g in range(CHUNK // 16):

            def row_step(r, accs):
                b = g * 16 + r
                sel = lane == r
                c = [csum_v[b, pl.ds(k * 16, 16)] for k in range(K16)]
                w = [w_v[b, pl.ds(k * 16, 16)] for k in range(K16)]
                prod = c[0] * w[0]
                for k in range(1, K16):
                    prod += c[k] * w[k]
                new = [jnp.where(sel, jnp.sum(prod), accs[0])]
                for n in range(NEG):
                    a = c[0] * neg_v[b * NEG + n, pl.ds(0, 16)]
                    for k in range(1, K16):
                        a += c[k] * neg_v[b * NEG + n, pl.ds(k * 16, 16)]
                    new.append(jnp.where(sel, jnp.sum(a), accs[n + 1]))
                return tuple(new)

            accs = lax.fori_loop(0, 16, row_step, (zero,) * (NEG + 1))
            pos_o[pl.ds(g * 16, 16)] = accs[0]
            for n in range(NEG):
                neg_o[n, pl.ds(g * 16, 16)] = accs[n + 1]

        pltpu.sync_copy(pos_o, pos_hbm.at[pl.ds(base, CHUNK)])
        pltpu.sync_copy(neg_o, neg_hbm.at[:, pl.ds(base, CHUNK)])


_sc_ips = functools.partial(
    pl.kernel,
    out_type=(
        jax.ShapeDtypeStruct((B,), jnp.float32),
        jax.ShapeDtypeStruct((NEG, B), jnp.float32),
    ),
    mesh=plsc.VectorSubcoreMesh(
        core_axis_name="c", subcore_axis_name="s",
        num_cores=NC, num_subcores=NS),
    scratch_types=[
        pltpu.VMEM((L * CHUNK,), jnp.int32),
        pltpu.VMEM((L, CHUNK), jnp.int32),
        pltpu.VMEM((CHUNK,), jnp.int32),
        pltpu.VMEM((NEG * CHUNK,), jnp.int32),
        pltpu.VMEM((CHUNK, DIM), jnp.float32),
        pltpu.VMEM((CHUNK, DIM), jnp.float32),
        pltpu.VMEM((NEG * CHUNK, DIM), jnp.float32),
        pltpu.VMEM((CHUNK,), jnp.float32),
        pltpu.VMEM((NEG, CHUNK), jnp.float32),
        pltpu.SemaphoreType.DMA,
        pltpu.SemaphoreType.DMA,
        pltpu.SemaphoreType.DMA,
    ],
    compiler_params=pltpu.CompilerParams(
        needs_layout_passes=False, use_tc_tiling_on_sc=False),
)(_sc_body)


def _loss_body(pos_ref, neg_ref, lens_ref, out_ref):
    inv = 1.0 / lens_ref[...]
    zp = jnp.clip(pos_ref[...] * inv, -10.0, 10.0)
    zn = jnp.clip(-(neg_ref[...] * inv[None]), -10.0, 10.0)
    out_ref[0, 0] = (jnp.sum(-jax.nn.log_sigmoid(zp)) +
                     jnp.sum(-jax.nn.log_sigmoid(zn)))


_loss = pl.pallas_call(
    _loss_body,
    out_shape=jax.ShapeDtypeStruct((1, 1), jnp.float32),
    out_specs=pl.BlockSpec(memory_space=pltpu.SMEM),
)


def kernel(word_idx, ctx_inds, ctx_lens, neg_inds, emb0_weight, emb1_weight):
    ctx_flat = ctx_inds.astype(jnp.int32).reshape(-1)
    neg_flat = neg_inds.astype(jnp.int32).reshape(-1)
    word = word_idx.astype(jnp.int32)
    pos_ips, neg_ips = _sc_ips(ctx_flat, word, neg_flat,
                               emb0_weight, emb1_weight)
    out = _loss(pos_ips.reshape(128, 128),
                neg_ips.reshape(NEG, 128, 128),
                ctx_lens.reshape(128, 128))
    return out[0, 0]


# speedup vs baseline: 1.1166x; 1.1166x over previous
import functools

import jax
import jax.numpy as jnp
from jax import lax
from jax.experimental import pallas as pl
from jax.experimental.pallas import tpu as pltpu
from jax.experimental.pallas import tpu_sc as plsc

NC = 2
NS = 16
NW = NC * NS

VOCAB = 100000
DIM = 64
B = 16384
L = 10
NEG = 5

CHUNK = 128
ROWS_PER_W = B // NW
NCHUNK = ROWS_PER_W // CHUNK
K16 = DIM // 16

_MESH = plsc.VectorSubcoreMesh(
    core_axis_name="c", subcore_axis_name="s",
    num_cores=NC, num_subcores=NS)
_PARAMS = pltpu.CompilerParams(
    needs_layout_passes=False, use_tc_tiling_on_sc=False)


def _ctx_body(ctx_t, emb0, csum_hbm, ctx_v, csum_v, sem_c):
    wid = lax.axis_index("s") * NC + lax.axis_index("c")
    for chunk in range(NCHUNK):
        base = wid * ROWS_PER_W + chunk * CHUNK
        pltpu.sync_copy(ctx_t.at[:, pl.ds(base, CHUNK)], ctx_v)
        pltpu.async_copy(emb0.at[ctx_v.at[0]], csum_v, sem_c).wait()
        d_c = [pltpu.async_copy(emb0.at[ctx_v.at[l]], csum_v, sem_c,
                                add=True)
               for l in range(1, L)]
        for d in d_c:
            d.wait()
        pltpu.sync_copy(csum_v, csum_hbm.at[pl.ds(base, CHUNK)])


_sc_ctx = functools.partial(
    pl.kernel,
    out_type=jax.ShapeDtypeStruct((B, DIM), jnp.float32),
    mesh=_MESH,
    scratch_types=[
        pltpu.VMEM((L, CHUNK), jnp.int32),
        pltpu.VMEM((CHUNK, DIM), jnp.float32),
        pltpu.SemaphoreType.DMA,
    ],
    compiler_params=_PARAMS,
)(_ctx_body)


def _ips_body(word, neg_t, emb1, csum_hbm, pos_hbm, neg_hbm,
              widx_v, nidx_v, csum_v, w_v, neg_v, pos_o, neg_o,
              sem_c, sem_w, sem_n):
    wid = lax.axis_index("s") * NC + lax.axis_index("c")
    lane = lax.iota(jnp.int32, 16)
    for chunk in range(NCHUNK):
        base = wid * ROWS_PER_W + chunk * CHUNK
        pltpu.sync_copy(word.at[pl.ds(base, CHUNK)], widx_v)
        pltpu.sync_copy(neg_t.at[:, pl.ds(base, CHUNK)], nidx_v)
        d_c = pltpu.async_copy(csum_hbm.at[pl.ds(base, CHUNK)], csum_v, sem_c)
        d_w = pltpu.async_copy(emb1.at[widx_v], w_v, sem_w)
        d_n = [pltpu.async_copy(emb1.at[nidx_v.at[n]],
                                neg_v.at[pl.ds(n * CHUNK, CHUNK)], sem_n)
               for n in range(NEG)]
        d_c.wait()
        d_w.wait()
        for d in d_n:
            d.wait()

        zero = jnp.zeros((16,), jnp.float32)
        for g in range(CHUNK // 16):

            def row_step(r, accs):
                b = g * 16 + r
                sel = lane == r
                c = [csum_v[b, pl.ds(k * 16, 16)] for k in range(K16)]
                w = [w_v[b, pl.ds(k * 16, 16)] for k in range(K16)]
                prod = c[0] * w[0]
                for k in range(1, K16):
                    prod += c[k] * w[k]
                new = [jnp.where(sel, jnp.sum(prod), accs[0])]
                for n in range(NEG):
                    a = c[0] * neg_v[n * CHUNK + b, pl.ds(0, 16)]
                    for k in range(1, K16):
                        a += c[k] * neg_v[n * CHUNK + b, pl.ds(k * 16, 16)]
                    new.append(jnp.where(sel, jnp.sum(a), accs[n + 1]))
                return tuple(new)

            accs = lax.fori_loop(0, 16, row_step, (zero,) * (NEG + 1))
            pos_o[pl.ds(g * 16, 16)] = accs[0]
            for n in range(NEG):
                neg_o[n, pl.ds(g * 16, 16)] = accs[n + 1]

        pltpu.sync_copy(pos_o, pos_hbm.at[pl.ds(base, CHUNK)])
        pltpu.sync_copy(neg_o, neg_hbm.at[:, pl.ds(base, CHUNK)])


_sc_ips = functools.partial(
    pl.kernel,
    out_type=(
        jax.ShapeDtypeStruct((B,), jnp.float32),
        jax.ShapeDtypeStruct((NEG, B), jnp.float32),
    ),
    mesh=_MESH,
    scratch_types=[
        pltpu.VMEM((CHUNK,), jnp.int32),
        pltpu.VMEM((NEG, CHUNK), jnp.int32),
        pltpu.VMEM((CHUNK, DIM), jnp.float32),
        pltpu.VMEM((CHUNK, DIM), jnp.float32),
        pltpu.VMEM((NEG * CHUNK, DIM), jnp.float32),
        pltpu.VMEM((CHUNK,), jnp.float32),
        pltpu.VMEM((NEG, CHUNK), jnp.float32),
        pltpu.SemaphoreType.DMA,
        pltpu.SemaphoreType.DMA,
        pltpu.SemaphoreType.DMA,
    ],
    compiler_params=_PARAMS,
)(_ips_body)


def _loss_body(pos_ref, neg_ref, lens_ref, out_ref):
    inv = 1.0 / lens_ref[...]
    zp = jnp.clip(pos_ref[...] * inv, -10.0, 10.0)
    zn = jnp.clip(-(neg_ref[...] * inv[None]), -10.0, 10.0)
    out_ref[0, 0] = (jnp.sum(-jax.nn.log_sigmoid(zp)) +
                     jnp.sum(-jax.nn.log_sigmoid(zn)))


_loss = pl.pallas_call(
    _loss_body,
    out_shape=jax.ShapeDtypeStruct((1, 1), jnp.float32),
    out_specs=pl.BlockSpec(memory_space=pltpu.SMEM),
)


def kernel(word_idx, ctx_inds, ctx_lens, neg_inds, emb0_weight, emb1_weight):
    ctx_t = jnp.transpose(ctx_inds).astype(jnp.int32)
    neg_t = jnp.transpose(neg_inds).astype(jnp.int32)
    word = word_idx.astype(jnp.int32)
    csum = _sc_ctx(ctx_t, emb0_weight)
    pos_ips, neg_ips = _sc_ips(word, neg_t, emb1_weight, csum)
    out = _loss(pos_ips.reshape(128, 128),
                neg_ips.reshape(NEG, 128, 128),
                ctx_lens.reshape(128, 128))
    return out[0, 0]


# speedup vs baseline: 1.1852x; 1.0614x over previous
import functools

import jax
import jax.numpy as jnp
from jax import lax
from jax.experimental import pallas as pl
from jax.experimental.pallas import tpu as pltpu
from jax.experimental.pallas import tpu_sc as plsc

NC = 2
NS = 16
NW = NC * NS

VOCAB = 100000
DIM = 64
B = 16384
L = 10
NEG = 5

CHUNK = 128
ROWS_PER_W = B // NW
NCHUNK = ROWS_PER_W // CHUNK
K16 = DIM // 16

_MESH = plsc.VectorSubcoreMesh(
    core_axis_name="c", subcore_axis_name="s",
    num_cores=NC, num_subcores=NS)
_PARAMS = pltpu.CompilerParams(
    needs_layout_passes=False, use_tc_tiling_on_sc=False)


def _ctx_body(ctx_t, emb0, csum_hbm, ctx_v, csum_v, sem_l0, sem_add, sem_out):
    wid = lax.axis_index("s") * NC + lax.axis_index("c")
    base0 = wid * ROWS_PER_W
    pltpu.sync_copy(ctx_t.at[:, pl.ds(base0, ROWS_PER_W)], ctx_v)
    d0 = [pltpu.async_copy(emb0.at[ctx_v.at[0, pl.ds(i * CHUNK, CHUNK)]],
                           csum_v.at[i], sem_l0)
          for i in range(NCHUNK)]
    for d in d0:
        d.wait()
    adds = [pltpu.async_copy(emb0.at[ctx_v.at[l, pl.ds(i * CHUNK, CHUNK)]],
                             csum_v.at[i], sem_add, add=True)
            for i in range(NCHUNK) for l in range(1, L)]
    for d in adds:
        d.wait()
    outs = [pltpu.async_copy(csum_v.at[i],
                             csum_hbm.at[pl.ds(base0 + i * CHUNK, CHUNK)],
                             sem_out)
            for i in range(NCHUNK)]
    for d in outs:
        d.wait()


_sc_ctx = functools.partial(
    pl.kernel,
    out_type=jax.ShapeDtypeStruct((B, DIM), jnp.float32),
    mesh=_MESH,
    scratch_types=[
        pltpu.VMEM((L, ROWS_PER_W), jnp.int32),
        pltpu.VMEM((NCHUNK, CHUNK, DIM), jnp.float32),
        pltpu.SemaphoreType.DMA,
        pltpu.SemaphoreType.DMA,
        pltpu.SemaphoreType.DMA,
    ],
    compiler_params=_PARAMS,
)(_ctx_body)


def _ips_body(word, neg_t, emb1, csum_hbm, pos_hbm, neg_hbm,
              widx_v, nidx_v, csum_v, w_v, neg_v, pos_o, neg_o,
              sem_b0, sem_b1, sem_o0, sem_o1):
    wid = lax.axis_index("s") * NC + lax.axis_index("c")
    lane = lax.iota(jnp.int32, 16)
    base0 = wid * ROWS_PER_W
    sem_b = [sem_b0, sem_b1]
    sem_o = [sem_o0, sem_o1]

    def load(i, buf):
        base = base0 + i * CHUNK
        pltpu.sync_copy(word.at[pl.ds(base, CHUNK)], widx_v.at[buf])
        pltpu.sync_copy(neg_t.at[:, pl.ds(base, CHUNK)], nidx_v.at[buf])
        d = [pltpu.async_copy(csum_hbm.at[pl.ds(base, CHUNK)],
                              csum_v.at[buf], sem_b[buf]),
             pltpu.async_copy(emb1.at[widx_v.at[buf]], w_v.at[buf],
                              sem_b[buf])]
        d += [pltpu.async_copy(emb1.at[nidx_v.at[buf, n]],
                               neg_v.at[buf, pl.ds(n * CHUNK, CHUNK)],
                               sem_b[buf])
              for n in range(NEG)]
        return d

    descs = load(0, 0)
    outs = []
    for i in range(NCHUNK):
        cur = i % 2
        if i + 1 < NCHUNK:
            next_descs = load(i + 1, 1 - cur)
        for d in descs:
            d.wait()
        if i >= 2:
            for d in outs[i - 2]:
                d.wait()

        zero = jnp.zeros((16,), jnp.float32)
        for g in range(CHUNK // 16):

            def row_step(r, accs):
                b = g * 16 + r
                sel = lane == r
                c = [csum_v[cur, b, pl.ds(k * 16, 16)] for k in range(K16)]
                w = [w_v[cur, b, pl.ds(k * 16, 16)] for k in range(K16)]
                prod = c[0] * w[0]
                for k in range(1, K16):
                    prod += c[k] * w[k]
                new = [jnp.where(sel, jnp.sum(prod), accs[0])]
                for n in range(NEG):
                    a = c[0] * neg_v[cur, n * CHUNK + b, pl.ds(0, 16)]
                    for k in range(1, K16):
                        a += c[k] * neg_v[cur, n * CHUNK + b,
                                          pl.ds(k * 16, 16)]
                    new.append(jnp.where(sel, jnp.sum(a), accs[n + 1]))
                return tuple(new)

            accs = lax.fori_loop(0, 16, row_step, (zero,) * (NEG + 1))
            pos_o[cur, pl.ds(g * 16, 16)] = accs[0]
            for n in range(NEG):
                neg_o[cur, n, pl.ds(g * 16, 16)] = accs[n + 1]

        base = base0 + i * CHUNK
        outs.append([
            pltpu.async_copy(pos_o.at[cur], pos_hbm.at[pl.ds(base, CHUNK)],
                             sem_o[cur]),
            pltpu.async_copy(neg_o.at[cur], neg_hbm.at[:, pl.ds(base, CHUNK)],
                             sem_o[cur])])
        if i + 1 < NCHUNK:
            descs = next_descs
    for d in outs[-2] + outs[-1]:
        d.wait()


_sc_ips = functools.partial(
    pl.kernel,
    out_type=(
        jax.ShapeDtypeStruct((B,), jnp.float32),
        jax.ShapeDtypeStruct((NEG, B), jnp.float32),
    ),
    mesh=_MESH,
    scratch_types=[
        pltpu.VMEM((2, CHUNK), jnp.int32),
        pltpu.VMEM((2, NEG, CHUNK), jnp.int32),
        pltpu.VMEM((2, CHUNK, DIM), jnp.float32),
        pltpu.VMEM((2, CHUNK, DIM), jnp.float32),
        pltpu.VMEM((2, NEG * CHUNK, DIM), jnp.float32),
        pltpu.VMEM((2, CHUNK), jnp.float32),
        pltpu.VMEM((2, NEG, CHUNK), jnp.float32),
        pltpu.SemaphoreType.DMA,
        pltpu.SemaphoreType.DMA,
        pltpu.SemaphoreType.DMA,
        pltpu.SemaphoreType.DMA,
    ],
    compiler_params=_PARAMS,
)(_ips_body)


def _loss_body(pos_ref, neg_ref, lens_ref, out_ref):
    inv = 1.0 / lens_ref[...]
    zp = jnp.clip(pos_ref[...] * inv, -10.0, 10.0)
    zn = jnp.clip(-(neg_ref[...] * inv[None]), -10.0, 10.0)
    out_ref[0, 0] = (jnp.sum(-jax.nn.log_sigmoid(zp)) +
                     jnp.sum(-jax.nn.log_sigmoid(zn)))


_loss = pl.pallas_call(
    _loss_body,
    out_shape=jax.ShapeDtypeStruct((1, 1), jnp.float32),
    out_specs=pl.BlockSpec(memory_space=pltpu.SMEM),
)


def kernel(word_idx, ctx_inds, ctx_lens, neg_inds, emb0_weight, emb1_weight):
    ctx_t = jnp.transpose(ctx_inds).astype(jnp.int32)
    neg_t = jnp.transpose(neg_inds).astype(jnp.int32)
    word = word_idx.astype(jnp.int32)
    csum = _sc_ctx(ctx_t, emb0_weight)
    pos_ips, neg_ips = _sc_ips(word, neg_t, emb1_weight, csum)
    out = _loss(pos_ips.reshape(128, 128),
                neg_ips.reshape(NEG, 128, 128),
                ctx_lens.reshape(128, 128))
    return out[0, 0]
